# baseline (speedup 1.0000x reference)
import functools

import jax
import jax.numpy as jnp
from jax import lax
from jax.experimental import pallas as pl
from jax.experimental.pallas import tpu as pltpu
from jax.experimental.pallas import tpu_sc as plsc

N_NODES = 10000
D_FEAT = 128
N_EDGES = 320000

NC = 2
NS = 16
NW = NC * NS

CHUNK = 80
RING = 4
CHUNKS_PER_W = 128
ROUNDS = CHUNKS_PER_W // RING
PER_W = CHUNK * CHUNKS_PER_W
E_PAD = PER_W * NW

ACC_ROWS = 10240
ROWS_PER_TILE = ACC_ROWS // NS
DUMMY_ROW = N_NODES

_mesh = plsc.VectorSubcoreMesh(core_axis_name="c", subcore_axis_name="s")


@functools.partial(
    pl.kernel,
    mesh=_mesh,
    out_type=jax.ShapeDtypeStruct((NC, ACC_ROWS, D_FEAT), jnp.float32),
    scratch_types=[
        pltpu.VMEM_SHARED((ACC_ROWS, D_FEAT), jnp.float32),
        pltpu.VMEM((RING, CHUNK), jnp.int32),
        pltpu.VMEM((RING, CHUNK), jnp.int32),
        pltpu.VMEM((RING, CHUNK, D_FEAT), jnp.float32),
        pltpu.SemaphoreType.DMA,
        pltpu.SemaphoreType.DMA,
    ],
)
def _sc_gather_scatter(x_hbm, src_hbm, dst_hbm, zeros_hbm, part_hbm,
                       acc, src_v, dst_v, rows_v, gsem, ssem):
    c = lax.axis_index("c")
    s = lax.axis_index("s")
    wid = s * NC + c

    for b in range(RING - 1):
        pltpu.sync_copy(src_hbm.at[wid, b], src_v.at[b])
        pltpu.sync_copy(dst_hbm.at[wid, b], dst_v.at[b])
        pltpu.async_copy(x_hbm.at[src_v.at[b]], rows_v.at[b], gsem)

    pltpu.sync_copy(zeros_hbm.at[pl.ds(s * ROWS_PER_TILE, ROWS_PER_TILE)],
                    acc.at[pl.ds(s * ROWS_PER_TILE, ROWS_PER_TILE)])
    plsc.subcore_barrier()

    def round_body(r, carry):
        for b in range(RING):
            g = r * RING + b
            pltpu.make_async_copy(x_hbm.at[src_v.at[b]], rows_v.at[b],
                                  gsem).wait()
            bn = (b + RING - 1) % RING
            g2 = g + RING - 1

            def stage():
                pltpu.sync_copy(src_hbm.at[wid, g2], src_v.at[bn])
                pltpu.sync_copy(dst_hbm.at[wid, g2], dst_v.at[bn])
                pltpu.async_copy(x_hbm.at[src_v.at[bn]], rows_v.at[bn], gsem)

            if (ROUNDS - 1) * RING + b + RING - 1 < CHUNKS_PER_W:
                stage()
            else:
                pl.when(g2 < CHUNKS_PER_W)(stage)
        return carry

    lax.fori_loop(0, ROUNDS, round_body, 0)

    plsc.subcore_barrier()
    pltpu.sync_copy(acc.at[pl.ds(s * ROWS_PER_TILE, ROWS_PER_TILE)],
                    part_hbm.at[c, pl.ds(s * ROWS_PER_TILE, ROWS_PER_TILE)])


def _add_body(a_ref, b_ref, o_ref):
    o_ref[...] = a_ref[0] + b_ref[0]


_ADD_ROWS = 400


def _combine_partials(part):
    return pl.pallas_call(
        _add_body,
        out_shape=jax.ShapeDtypeStruct((N_NODES, D_FEAT), jnp.float32),
        grid=(N_NODES // _ADD_ROWS,),
        in_specs=[
            pl.BlockSpec((1, _ADD_ROWS, D_FEAT), lambda i: (0, i, 0)),
            pl.BlockSpec((1, _ADD_ROWS, D_FEAT), lambda i: (1, i, 0)),
        ],
        out_specs=pl.BlockSpec((_ADD_ROWS, D_FEAT), lambda i: (i, 0)),
    )(part, part)


def kernel(x, edge_index):
    pad = E_PAD - N_EDGES
    src = jnp.concatenate([edge_index[0], jnp.zeros((pad,), jnp.int32)])
    dst = jnp.concatenate(
        [edge_index[1], jnp.full((pad,), DUMMY_ROW, jnp.int32)])
    src = src.reshape(NW, CHUNKS_PER_W, CHUNK)
    dst = dst.reshape(NW, CHUNKS_PER_W, CHUNK)
    zeros = jnp.zeros((ACC_ROWS, D_FEAT), jnp.float32)
    part = _sc_gather_scatter(x, src, dst, zeros)
    return _combine_partials(part)


# speedup vs baseline: 1.9183x; 1.9183x over previous
import functools

import jax
import jax.numpy as jnp
from jax import lax
from jax.experimental import pallas as pl
from jax.experimental.pallas import tpu as pltpu
from jax.experimental.pallas import tpu_sc as plsc

N_NODES = 10000
D_FEAT = 128
N_EDGES = 320000

NC = 2
NS = 16
NW = NC * NS

CHUNK = 112
RING = 3
IR = 4
CHUNKS_PER_W = 90
ROUNDS = CHUNKS_PER_W // RING
PER_W = CHUNK * CHUNKS_PER_W
E_PAD = PER_W * NW

ACC_ROWS = 10240
ROWS_PER_TILE = ACC_ROWS // NS
DUMMY_ROW = N_NODES

_mesh = plsc.VectorSubcoreMesh(core_axis_name="c", subcore_axis_name="s")


@functools.partial(
    pl.kernel,
    mesh=_mesh,
    out_type=jax.ShapeDtypeStruct((NC, ACC_ROWS, D_FEAT), jnp.float32),
    scratch_types=[
        pltpu.VMEM_SHARED((ACC_ROWS, D_FEAT), jnp.float32),
        pltpu.VMEM((IR, 2, CHUNK), jnp.int32),
        pltpu.VMEM((RING, CHUNK, D_FEAT), jnp.float32),
        pltpu.SemaphoreType.DMA,
        pltpu.SemaphoreType.DMA,
        pltpu.SemaphoreType.DMA,
    ],
)
def _sc_gather_scatter(x_hbm, idx_hbm, zeros_hbm, part_hbm,
                       acc, idx_v, rows_v, isem, gsem, ssem):
    c = lax.axis_index("c")
    s = lax.axis_index("s")
    wid = s * NC + c

    for b in range(RING - 1):
        pltpu.sync_copy(idx_hbm.at[wid, b], idx_v.at[b])
        pltpu.async_copy(x_hbm.at[idx_v.at[b, 0]], rows_v.at[b], gsem)
    pltpu.async_copy(idx_hbm.at[wid, RING - 1], idx_v.at[RING - 1], isem)

    pltpu.sync_copy(zeros_hbm.at[pl.ds(s * ROWS_PER_TILE, ROWS_PER_TILE)],
                    acc.at[pl.ds(s * ROWS_PER_TILE, ROWS_PER_TILE)])
    plsc.subcore_barrier()

    def round_body(r, carry):
        for b in range(RING):
            g = r * RING + b
            pltpu.make_async_copy(x_hbm.at[idx_v.at[g % IR, 0]],
                                  rows_v.at[b], gsem).wait()
            pltpu.async_copy(rows_v.at[b], acc.at[idx_v.at[g % IR, 1]], ssem,
                             add=True)
            bp = (b - 1) % RING
            drain = pltpu.make_async_copy(
                rows_v.at[bp], acc.at[idx_v.at[(g - 1) % IR, 1]], ssem)
            if b == 0:
                @pl.when(r > 0)
                def _():
                    drain.wait()
            else:
                drain.wait()

            g2 = g + RING - 1
            g3 = g + RING

            def stage():
                pltpu.make_async_copy(idx_hbm.at[wid, g2],
                                      idx_v.at[g2 % IR], isem).wait()
                pltpu.async_copy(x_hbm.at[idx_v.at[g2 % IR, 0]],
                                 rows_v.at[bp], gsem)

            def prefetch():
                pltpu.async_copy(idx_hbm.at[wid, g3], idx_v.at[g3 % IR], isem)

            if (ROUNDS - 1) * RING + b + RING - 1 < CHUNKS_PER_W:
                stage()
            else:
                pl.when(g2 < CHUNKS_PER_W)(stage)
            if (ROUNDS - 1) * RING + b + RING < CHUNKS_PER_W:
                prefetch()
            else:
                pl.when(g3 < CHUNKS_PER_W)(prefetch)
        return carry

    lax.fori_loop(0, ROUNDS, round_body, 0)

    pltpu.make_async_copy(rows_v.at[RING - 1],
                          acc.at[idx_v.at[(CHUNKS_PER_W - 1) % IR, 1]],
                          ssem).wait()

    plsc.subcore_barrier()
    pltpu.sync_copy(acc.at[pl.ds(s * ROWS_PER_TILE, ROWS_PER_TILE)],
                    part_hbm.at[c, pl.ds(s * ROWS_PER_TILE, ROWS_PER_TILE)])


def _add_body(a_ref, b_ref, o_ref):
    o_ref[...] = a_ref[0] + b_ref[0]


_ADD_ROWS = 400


def _combine_partials(part):
    return pl.pallas_call(
        _add_body,
        out_shape=jax.ShapeDtypeStruct((N_NODES, D_FEAT), jnp.float32),
        grid=(N_NODES // _ADD_ROWS,),
        in_specs=[
            pl.BlockSpec((1, _ADD_ROWS, D_FEAT), lambda i: (0, i, 0)),
            pl.BlockSpec((1, _ADD_ROWS, D_FEAT), lambda i: (1, i, 0)),
        ],
        out_specs=pl.BlockSpec((_ADD_ROWS, D_FEAT), lambda i: (i, 0)),
    )(part, part)


def kernel(x, edge_index):
    pad = E_PAD - N_EDGES
    src = jnp.concatenate([edge_index[0], jnp.zeros((pad,), jnp.int32)])
    dst = jnp.concatenate(
        [edge_index[1], jnp.full((pad,), DUMMY_ROW, jnp.int32)])
    idx = jnp.stack([src.reshape(NW, CHUNKS_PER_W, CHUNK),
                     dst.reshape(NW, CHUNKS_PER_W, CHUNK)], axis=2)
    zeros = jnp.zeros((ACC_ROWS, D_FEAT), jnp.float32)
    part = _sc_gather_scatter(x, idx, zeros)
    return _combine_partials(part)
